# baseline (speedup 1.0000x reference)
import functools

import jax
import jax.numpy as jnp
from jax import lax
from jax.experimental import pallas as pl
from jax.experimental.pallas import tpu as pltpu
from jax.experimental.pallas import tpu_sc as plsc

V = 10000
D = 128
B = 320000

_info = plsc.get_sparse_core_info()
NC, NS = _info.num_cores, _info.num_subcores
NW = NC * NS
B_PER_W = B // NW
C = 80
N_CHUNKS = B_PER_W // C
NBUF = 4
G = (N_CHUNKS - 1) // NBUF
PEEL = 3
SKEW = 2

_mesh = plsc.VectorSubcoreMesh(core_axis_name="c", subcore_axis_name="s")


@functools.partial(
    pl.kernel,
    mesh=_mesh,
    out_type=jax.ShapeDtypeStruct((B, D), jnp.float32),
    scratch_types=[
        pltpu.VMEM((NBUF, C), jnp.int32),
        pltpu.VMEM((NBUF, C, D), jnp.float32),
        pltpu.VMEM_SHARED((V, D), jnp.float32),
        pltpu.SemaphoreType.DMA((NBUF,)),
        pltpu.SemaphoreType.DMA((NBUF,)),
        pltpu.SemaphoreType.DMA((NBUF,)),
        pltpu.SemaphoreType.DMA,
    ],
)
def _gather_sc(x_hbm, idx_hbm, out_hbm, idx_v, rows_v, xs, isem, gsem, ssem,
               xsem):
    sid = lax.axis_index("s")
    wid = sid * NC + lax.axis_index("c")
    base_w = wid * B_PER_W

    RS = 632

    def stage_copy(n):
        r0 = pl.multiple_of(sid * RS, 8) if n == RS else (NS - 1) * RS
        return pltpu.make_async_copy(
            x_hbm.at[pl.ds(r0, n)], xs.at[pl.ds(r0, n)], xsem)

    @pl.when(sid < NS - 1)
    def _():
        stage_copy(RS).start()

    @pl.when(sid == NS - 1)
    def _():
        stage_copy(V - (NS - 1) * RS).start()

    def idx_copy(i, b):
        off = pl.multiple_of(B + base_w + i * C, 8)
        return pltpu.make_async_copy(
            idx_hbm.at[pl.ds(off, C)], idx_v.at[b], isem.at[b])

    def gather_copy(b, src):
        return pltpu.make_async_copy(
            src.at[idx_v.at[b]], rows_v.at[b], gsem.at[b])

    def store_copy(i, b):
        off = pl.multiple_of(base_w + i * C, 8)
        return pltpu.make_async_copy(
            rows_v.at[b], out_hbm.at[pl.ds(off, C)], ssem.at[b])

    idx_copy(0, 0).start()
    idx_copy(1, 1).start()

    def ring_stage(g, b, src, first_round):
        i = g * NBUF + b
        if not first_round:
            @pl.when(True if isinstance(g, int) else g > 0)
            def _():
                store_copy(0, b).wait()

        pb = (b - SKEW) % NBUF
        if b >= SKEW:
            gather_copy(pb, src).wait()
            store_copy(i - SKEW, pb).start()
        elif not first_round:
            gather_copy(pb, src).wait()
            store_copy(i - SKEW, pb).start()

        @pl.when(i <= N_CHUNKS - 1 - SKEW if not isinstance(i, int)
                 else jnp.bool_(i <= N_CHUNKS - 1 - SKEW))
        def _():
            idx_copy(i + SKEW, (b + SKEW) % NBUF).start()

        idx_copy(0, b).wait()
        gather_copy(b, src).start()

    for g in range(PEEL):
        for b in range(NBUF):
            ring_stage(g, b, x_hbm, first_round=(g == 0))

    @pl.when(sid < NS - 1)
    def _():
        stage_copy(RS).wait()

    @pl.when(sid == NS - 1)
    def _():
        stage_copy(V - (NS - 1) * RS).wait()
    plsc.subcore_barrier()

    def outer(g, carry):
        for b in range(NBUF):
            ring_stage(g, b, xs, first_round=False)
        return carry

    lax.fori_loop(PEEL, G, outer, 0)

    gather_copy(2, xs).wait()
    store_copy(N_CHUNKS - 3, 2).start()
    store_copy(0, 0).wait()
    idx_copy(0, 0).wait()
    gather_copy(0, xs).start()
    gather_copy(3, xs).wait()
    store_copy(N_CHUNKS - 2, 3).start()
    gather_copy(0, xs).wait()
    store_copy(N_CHUNKS - 1, 0).start()
    for b in range(1, NBUF):
        store_copy(0, b).wait()
    store_copy(0, 0).wait()


def kernel(x, edge_index):
    return _gather_sc(x, edge_index.reshape(-1))


# speedup vs baseline: 1.0224x; 1.0224x over previous
import functools

import jax
import jax.numpy as jnp
from jax import lax
from jax.experimental import pallas as pl
from jax.experimental.pallas import tpu as pltpu
from jax.experimental.pallas import tpu_sc as plsc

V = 10000
D = 128
B = 320000

_info = plsc.get_sparse_core_info()
NC, NS = _info.num_cores, _info.num_subcores
NW = NC * NS
B_PER_W = B // NW
C = 80
N_CHUNKS = B_PER_W // C
NBUF = 4
G = (N_CHUNKS - 1) // NBUF
PEEL = 2
SKEW = 2

_mesh = plsc.VectorSubcoreMesh(core_axis_name="c", subcore_axis_name="s")


@functools.partial(
    pl.kernel,
    mesh=_mesh,
    out_type=jax.ShapeDtypeStruct((B, D), jnp.float32),
    scratch_types=[
        pltpu.VMEM((NBUF, C), jnp.int32),
        pltpu.VMEM((NBUF, C, D), jnp.float32),
        pltpu.VMEM_SHARED((V, D), jnp.float32),
        pltpu.SemaphoreType.DMA((NBUF,)),
        pltpu.SemaphoreType.DMA((NBUF,)),
        pltpu.SemaphoreType.DMA((NBUF,)),
        pltpu.SemaphoreType.DMA,
    ],
)
def _gather_sc(x_hbm, idx_hbm, out_hbm, idx_v, rows_v, xs, isem, gsem, ssem,
               xsem):
    sid = lax.axis_index("s")
    wid = sid * NC + lax.axis_index("c")
    base_w = wid * B_PER_W

    RS = 632

    def stage_copy(n):
        r0 = pl.multiple_of(sid * RS, 8) if n == RS else (NS - 1) * RS
        return pltpu.make_async_copy(
            x_hbm.at[pl.ds(r0, n)], xs.at[pl.ds(r0, n)], xsem)

    @pl.when(sid < NS - 1)
    def _():
        stage_copy(RS).start()

    @pl.when(sid == NS - 1)
    def _():
        stage_copy(V - (NS - 1) * RS).start()

    def idx_copy(i, b):
        off = pl.multiple_of(B + base_w + i * C, 8)
        return pltpu.make_async_copy(
            idx_hbm.at[pl.ds(off, C)], idx_v.at[b], isem.at[b])

    def gather_copy(b, src):
        return pltpu.make_async_copy(
            src.at[idx_v.at[b]], rows_v.at[b], gsem.at[b])

    def store_copy(i, b):
        off = pl.multiple_of(base_w + i * C, 8)
        return pltpu.make_async_copy(
            rows_v.at[b], out_hbm.at[pl.ds(off, C)], ssem.at[b])

    idx_copy(0, 0).start()
    idx_copy(1, 1).start()

    def ring_stage(g, b, src, first_round):
        i = g * NBUF + b
        if not first_round:
            @pl.when(True if isinstance(g, int) else g > 0)
            def _():
                store_copy(0, b).wait()

        pb = (b - SKEW) % NBUF
        if b >= SKEW:
            gather_copy(pb, src).wait()
            store_copy(i - SKEW, pb).start()
        elif not first_round:
            gather_copy(pb, src).wait()
            store_copy(i - SKEW, pb).start()

        @pl.when(i <= N_CHUNKS - 1 - SKEW if not isinstance(i, int)
                 else jnp.bool_(i <= N_CHUNKS - 1 - SKEW))
        def _():
            idx_copy(i + SKEW, (b + SKEW) % NBUF).start()

        idx_copy(0, b).wait()
        gather_copy(b, src).start()

    for g in range(PEEL):
        for b in range(NBUF):
            ring_stage(g, b, x_hbm, first_round=(g == 0))

    @pl.when(sid < NS - 1)
    def _():
        stage_copy(RS).wait()

    @pl.when(sid == NS - 1)
    def _():
        stage_copy(V - (NS - 1) * RS).wait()
    plsc.subcore_barrier()

    def outer(g, carry):
        for b in range(NBUF):
            ring_stage(g, b, xs, first_round=False)
        return carry

    lax.fori_loop(PEEL, G, outer, 0)

    gather_copy(2, xs).wait()
    store_copy(N_CHUNKS - 3, 2).start()
    store_copy(0, 0).wait()
    idx_copy(0, 0).wait()
    gather_copy(0, xs).start()
    gather_copy(3, xs).wait()
    store_copy(N_CHUNKS - 2, 3).start()
    gather_copy(0, xs).wait()
    store_copy(N_CHUNKS - 1, 0).start()
    for b in range(1, NBUF):
        store_copy(0, b).wait()
    store_copy(0, 0).wait()


def kernel(x, edge_index):
    return _gather_sc(x, edge_index.reshape(-1))


# speedup vs baseline: 1.0234x; 1.0010x over previous
import functools

import jax
import jax.numpy as jnp
from jax import lax
from jax.experimental import pallas as pl
from jax.experimental.pallas import tpu as pltpu
from jax.experimental.pallas import tpu_sc as plsc

V = 10000
D = 128
B = 320000

_info = plsc.get_sparse_core_info()
NC, NS = _info.num_cores, _info.num_subcores
NW = NC * NS
B_PER_W = B // NW
C = 80
N_CHUNKS = B_PER_W // C
NBUF = 4
G = (N_CHUNKS - 1) // NBUF
PEEL = 2
SKEW = 2

_mesh = plsc.VectorSubcoreMesh(core_axis_name="c", subcore_axis_name="s")


@functools.partial(
    pl.kernel,
    mesh=_mesh,
    out_type=jax.ShapeDtypeStruct((B, D), jnp.float32),
    scratch_types=[
        pltpu.VMEM((B_PER_W,), jnp.int32),
        pltpu.VMEM((NBUF, C, D), jnp.float32),
        pltpu.VMEM_SHARED((V, D), jnp.float32),
        pltpu.SemaphoreType.DMA,
        pltpu.SemaphoreType.DMA((NBUF,)),
        pltpu.SemaphoreType.DMA((NBUF,)),
        pltpu.SemaphoreType.DMA,
    ],
)
def _gather_sc(x_hbm, idx_hbm, out_hbm, idx_v, rows_v, xs, isem, gsem, ssem,
               xsem):
    sid = lax.axis_index("s")
    wid = sid * NC + lax.axis_index("c")
    base_w = wid * B_PER_W

    RS = 632

    def stage_copy(n):
        r0 = pl.multiple_of(sid * RS, 8) if n == RS else (NS - 1) * RS
        return pltpu.make_async_copy(
            x_hbm.at[pl.ds(r0, n)], xs.at[pl.ds(r0, n)], xsem)

    @pl.when(sid < NS - 1)
    def _():
        stage_copy(RS).start()

    @pl.when(sid == NS - 1)
    def _():
        stage_copy(V - (NS - 1) * RS).start()

    idx_preload = pltpu.make_async_copy(
        idx_hbm.at[pl.ds(B + base_w, B_PER_W)], idx_v, isem)
    idx_preload.start()
    idx_preload.wait()

    def gather_copy(i, b, src):
        ioff = pl.multiple_of(i * C, 8)
        return pltpu.make_async_copy(
            src.at[idx_v.at[pl.ds(ioff, C)]], rows_v.at[b], gsem.at[b])

    def store_copy(i, b):
        off = pl.multiple_of(base_w + i * C, 8)
        return pltpu.make_async_copy(
            rows_v.at[b], out_hbm.at[pl.ds(off, C)], ssem.at[b])

    def ring_stage(g, b, src, first_round):
        i = g * NBUF + b
        if not first_round:
            store_copy(0, b).wait()

        pb = (b - SKEW) % NBUF
        if b >= SKEW or not first_round:
            gather_copy(0, pb, src).wait()
            store_copy(i - SKEW, pb).start()

        gather_copy(i, b, src).start()

    for g in range(PEEL):
        for b in range(NBUF):
            ring_stage(g, b, x_hbm, first_round=(g == 0))

    @pl.when(sid < NS - 1)
    def _():
        stage_copy(RS).wait()

    @pl.when(sid == NS - 1)
    def _():
        stage_copy(V - (NS - 1) * RS).wait()
    plsc.subcore_barrier()

    def outer(g, carry):
        for b in range(NBUF):
            ring_stage(g, b, xs, first_round=False)
        return carry

    lax.fori_loop(PEEL, G, outer, 0)

    gather_copy(0, 2, xs).wait()
    store_copy(N_CHUNKS - 3, 2).start()
    store_copy(0, 0).wait()
    gather_copy(N_CHUNKS - 1, 0, xs).start()
    gather_copy(0, 3, xs).wait()
    store_copy(N_CHUNKS - 2, 3).start()
    gather_copy(0, 0, xs).wait()
    store_copy(N_CHUNKS - 1, 0).start()
    for b in range(1, NBUF):
        store_copy(0, b).wait()
    store_copy(0, 0).wait()


def kernel(x, edge_index):
    return _gather_sc(x, edge_index.reshape(-1))
